# baseline (speedup 1.0000x reference)
import functools

import jax
import jax.numpy as jnp
from jax import lax
from jax.experimental import pallas as pl
from jax.experimental.pallas import tpu as pltpu
from jax.experimental.pallas import tpu_sc as plsc

_L = 16
_CH = 4


@jax.jit
def _gather(tbl, idx):
    info = plsc.get_sparse_core_info()
    nc, ns = info.num_cores, info.num_subcores
    nw = nc * ns
    batch = idx.shape[0]
    d = 32
    b_per_w = batch // nw
    n_pairs = b_per_w // _L

    idx_r = idx.reshape(nw, b_per_w, 2)

    mesh = plsc.VectorSubcoreMesh(core_axis_name="c", subcore_axis_name="s")

    @functools.partial(
        pl.kernel,
        mesh=mesh,
        out_type=jax.ShapeDtypeStruct((batch, d), jnp.float32),
        compiler_params=pltpu.CompilerParams(needs_layout_passes=False),
        scratch_types=[
            pltpu.VMEM((b_per_w, 2), jnp.int32),
            pltpu.VMEM((_CH, d, 128), jnp.float32),
            pltpu.VMEM((_CH, d, 128), jnp.float32),
            pltpu.VMEM((_L, d), jnp.float32),
            pltpu.SemaphoreType.DMA,
            pltpu.SemaphoreType.DMA,
            pltpu.SemaphoreType.DMA,
        ],
    )
    def k(tbl_hbm, idx_hbm, out_hbm,
          idx_v, buf0, buf1, outbuf, sem0, sem1, sem_o):
        wid = lax.axis_index("s") * nc + lax.axis_index("c")
        pltpu.sync_copy(idx_hbm.at[wid], idx_v)

        iota = lax.iota(jnp.int32, _L)

        def load_vecs(pair):
            row_v = iota + pair * _L
            t_vec = plsc.load_gather(idx_v, [row_v, jnp.zeros((_L,), jnp.int32)])
            r_vec = plsc.load_gather(idx_v, [row_v, jnp.ones((_L,), jnp.int32)])
            return t_vec, r_vec

        def issue(t_vec, r_vec, off, buf, sem):
            for j in range(_CH):
                t = t_vec[off + j]
                r = r_vec[off + j]
                col = pl.multiple_of((r >> 7) << 7, 128)
                row0 = pl.multiple_of(t * d, d)
                pltpu.async_copy(
                    tbl_hbm.at[pl.ds(row0, d), pl.ds(col, 128)],
                    buf.at[j], sem,
                )

        def drain(buf, sem):
            for j in range(_CH):
                pltpu.make_async_copy(
                    tbl_hbm.at[pl.ds(0, d), pl.ds(0, 128)], buf.at[j], sem
                ).wait()

        def select(r_vec, off, buf):
            lane_vec = r_vec & 127
            for j in range(_CH):
                lane_v = jnp.full((_L,), lane_vec[off + j], jnp.int32)
                j_v = jnp.full((_L,), j, jnp.int32)
                for h in range(d // _L):
                    c_v = iota + h * _L
                    vals = plsc.load_gather(buf, [j_v, c_v, lane_v])
                    outbuf[off + j, pl.ds(h * _L, _L)] = vals

        t0, r0 = load_vecs(0)
        issue(t0, r0, 0, buf0, sem0)
        issue(t0, r0, _CH, buf1, sem1)

        def body(kk, carry):
            t_vec, r_vec = carry
            nxt = jnp.minimum(kk + 1, n_pairs - 1)
            t_n, r_n = load_vecs(nxt)

            drain(buf0, sem0)
            select(r_vec, 0, buf0)
            issue(t_vec, r_vec, 2 * _CH, buf0, sem0)

            drain(buf1, sem1)
            select(r_vec, _CH, buf1)
            issue(t_vec, r_vec, 3 * _CH, buf1, sem1)

            drain(buf0, sem0)
            select(r_vec, 2 * _CH, buf0)

            @pl.when(kk + 1 < n_pairs)
            def _():
                issue(t_n, r_n, 0, buf0, sem0)

            drain(buf1, sem1)
            select(r_vec, 3 * _CH, buf1)

            @pl.when(kk + 1 < n_pairs)
            def _():
                issue(t_n, r_n, _CH, buf1, sem1)

            out_row = pl.multiple_of(wid * b_per_w + kk * _L, _L)
            pltpu.async_copy(
                outbuf, out_hbm.at[pl.ds(out_row, _L)], sem_o
            ).wait()
            return t_n, r_n

        lax.fori_loop(0, n_pairs, body, (t0, r0))

    return k(tbl, idx_r)


def kernel(train_table, indices):
    n_tables, vocab, d = train_table.shape
    tbl = jnp.transpose(train_table, (0, 2, 1)).reshape(n_tables * d, vocab)
    return _gather(tbl, indices.astype(jnp.int32))
